# baseline (speedup 1.0000x reference)
import functools

import numpy as np
import jax
import jax.numpy as jnp
from jax import lax
from jax.experimental import pallas as pl
from jax.experimental.pallas import tpu as pltpu
from jax.experimental.pallas import tpu_sc as plsc

B = 256
N = 512
OUT_LEN = N * (N + 1) // 2
GROUP_ROWS = 32
NGROUPS = N // GROUP_ROWS

NC, NS = 2, 16
NW = NC * NS
BATCH_PER_W = B // NW

_r, _c = np.triu_indices(N)
_g = _r // GROUP_ROWS

GLEN = [int(np.sum(_g == g)) for g in range(NGROUPS)]
GOFF = [int(np.searchsorted(_g, g)) for g in range(NGROUPS)]
PLEN = [-(-l // 64) * 64 for l in GLEN]
POFF = list(np.cumsum([0] + PLEN[:-1]))
LMAX = PLEN[0]
GWID = [N - GROUP_ROWS * g for g in range(NGROUPS)]

_local = (_r - _g * GROUP_ROWS) * N + (_c - _g * GROUP_ROWS)
_idx_parts = []
for _gg in range(NGROUPS):
    _part = _local[GOFF[_gg]:GOFF[_gg] + GLEN[_gg]]
    _idx_parts.append(np.pad(_part, (0, PLEN[_gg] - GLEN[_gg])))
IDX_LOCAL = np.concatenate(_idx_parts).astype(np.int32)


@functools.partial(
    pl.kernel,
    mesh=plsc.VectorSubcoreMesh(core_axis_name="c", subcore_axis_name="s"),
    out_type=jax.ShapeDtypeStruct((B * OUT_LEN,), jnp.float32),
    compiler_params=pltpu.CompilerParams(
        needs_layout_passes=False, use_tc_tiling_on_sc=False),
    scratch_types=[
        pltpu.VMEM((GROUP_ROWS, N), jnp.float32),
        pltpu.VMEM((GROUP_ROWS, N), jnp.float32),
        pltpu.VMEM((LMAX,), jnp.float32),
        pltpu.VMEM((LMAX,), jnp.float32),
        pltpu.VMEM((LMAX,), jnp.int32),
        pltpu.VMEM((LMAX,), jnp.int32),
        pltpu.SemaphoreType.DMA,
        pltpu.SemaphoreType.DMA,
        pltpu.SemaphoreType.DMA,
        pltpu.SemaphoreType.DMA,
        pltpu.SemaphoreType.DMA,
    ],
)
def _triu_sc(x_ref, idx_ref, out_ref, slab0, slab1, buf0, buf1, idxA, idxB,
             ss0, ss1, ts0, ts1, is0):
    wid = lax.axis_index("s") * NC + lax.axis_index("c")
    base = wid * BATCH_PER_W
    idxbufs = (idxA, idxB)

    def idx_cp(g, buf):
        return pltpu.make_async_copy(
            idx_ref.at[pl.ds(POFF[g], PLEN[g])], buf.at[pl.ds(0, PLEN[g])], is0)

    def slab_cp(g, b, sl, sem):
        w = GWID[g]
        return pltpu.make_async_copy(
            x_ref.at[b, pl.ds(g * GROUP_ROWS, GROUP_ROWS), pl.ds(g * GROUP_ROWS, w)],
            sl.at[:, pl.ds(0, w)], sem)

    idx_cp(0, idxA).start()
    idx_cp(0, idxA).wait()
    slab_cp(0, base, slab0, ss0).start()

    for g in range(NGROUPS):
        goff, glen, plen = GOFF[g], GLEN[g], PLEN[g]
        idxg = idxbufs[g % 2]

        def out_cp(b, buf, sem, goff=goff, glen=glen):
            return pltpu.make_async_copy(
                buf.at[pl.ds(0, glen)],
                out_ref.at[pl.ds(b * OUT_LEN + goff, glen)], sem)

        def gather(slab, buf, plen=plen, idxg=idxg):
            @plsc.parallel_loop(0, plen, 64, unroll=4)
            def _(o):
                for k in range(4):
                    ids = idxg[pl.ds(o + k * 16, 16)]
                    rows = lax.shift_right_logical(ids, 9)
                    cols = lax.bitwise_and(ids, N - 1)
                    buf[pl.ds(o + k * 16, 16)] = plsc.load_gather(
                        slab, [rows, cols])

        def pair_body(i2, _, g=g):
            i = i2 * 2
            ba = base + i
            slab_cp(g, ba, slab0, ss0).wait()
            slab_cp(g, ba + 1, slab1, ss1).start()

            @pl.when(i > 0)
            def _():
                out_cp(ba - 2, buf0, ts0).wait()

            gather(slab0, buf0)
            out_cp(ba, buf0, ts0).start()

            slab_cp(g, ba + 1, slab1, ss1).wait()

            @pl.when(i < BATCH_PER_W - 2)
            def _():
                slab_cp(g, ba + 2, slab0, ss0).start()

            @pl.when(i > 0)
            def _():
                out_cp(ba - 1, buf1, ts1).wait()

            gather(slab1, buf1)
            out_cp(ba + 1, buf1, ts1).start()
            return 0

        lax.fori_loop(0, BATCH_PER_W // 2, pair_body, 0, unroll=False)

        if g + 1 < NGROUPS:
            idx_cp(g + 1, idxbufs[(g + 1) % 2]).start()
            slab_cp(g + 1, base, slab0, ss0).start()
            idx_cp(g + 1, idxbufs[(g + 1) % 2]).wait()

        out_cp(base + BATCH_PER_W - 2, buf0, ts0).wait()
        out_cp(base + BATCH_PER_W - 1, buf1, ts1).wait()


_OFFR = [r * N - r * (r - 1) // 2 for r in range(N + 1)]


_TC_PANELS = [
    (0, 128, 0, 512),
    (128, 128, 128, 128),
    (128, 128, 256, 256),
    (256, 128, 256, 256),
    (384, 128, 384, 128),
]


def _tc_body(*refs):
    x_refs, out_ref = refs[:-1], refs[-1]
    for r in range(N - 1, -1, -1):
        for x_ref, (r0, nr, c0, nc) in zip(x_refs, _TC_PANELS):
            if not (r0 <= r < r0 + nr) or c0 + nc <= r:
                continue
            out_ref[0, 0, pl.ds(_OFFR[r] - r + c0, nc)] = x_ref[0, r - r0, :]


def _triu_tc(xs):
    nb = xs.shape[0]

    def spec(r0, nr, c0, nc):
        return pl.BlockSpec(
            (1, nr, nc), lambda b, r0=r0, nr=nr, c0=c0, nc=nc:
            (b, r0 // nr, c0 // nc))

    out = pl.pallas_call(
        _tc_body,
        grid=(nb,),
        in_specs=[spec(*p) for p in _TC_PANELS],
        out_specs=pl.BlockSpec((1, 1, OUT_LEN), lambda b: (b, 0, 0)),
        out_shape=jax.ShapeDtypeStruct((nb, 1, OUT_LEN), jnp.float32),
    )(*([xs] * len(_TC_PANELS)))
    return out.reshape(nb, OUT_LEN)


def kernel(x):
    return _triu_tc(x)


# speedup vs baseline: 1.2174x; 1.2174x over previous
import functools

import numpy as np
import jax
import jax.numpy as jnp
from jax import lax
from jax.experimental import pallas as pl
from jax.experimental.pallas import tpu as pltpu
from jax.experimental.pallas import tpu_sc as plsc

B = 256
N = 512
OUT_LEN = N * (N + 1) // 2
GROUP_ROWS = 32
NGROUPS = N // GROUP_ROWS

NC, NS = 2, 16
NW = NC * NS
BATCH_PER_W = B // NW

_r, _c = np.triu_indices(N)
_g = _r // GROUP_ROWS

GLEN = [int(np.sum(_g == g)) for g in range(NGROUPS)]
GOFF = [int(np.searchsorted(_g, g)) for g in range(NGROUPS)]
PLEN = [-(-l // 64) * 64 for l in GLEN]
POFF = list(np.cumsum([0] + PLEN[:-1]))
LMAX = PLEN[0]
GWID = [N - GROUP_ROWS * g for g in range(NGROUPS)]

_local = (_r - _g * GROUP_ROWS) * N + (_c - _g * GROUP_ROWS)
_idx_parts = []
for _gg in range(NGROUPS):
    _part = _local[GOFF[_gg]:GOFF[_gg] + GLEN[_gg]]
    _idx_parts.append(np.pad(_part, (0, PLEN[_gg] - GLEN[_gg])))
IDX_LOCAL = np.concatenate(_idx_parts).astype(np.int32)


@functools.partial(
    pl.kernel,
    mesh=plsc.VectorSubcoreMesh(core_axis_name="c", subcore_axis_name="s"),
    out_type=jax.ShapeDtypeStruct((B * OUT_LEN,), jnp.float32),
    compiler_params=pltpu.CompilerParams(
        needs_layout_passes=False, use_tc_tiling_on_sc=False),
    scratch_types=[
        pltpu.VMEM((GROUP_ROWS, N), jnp.float32),
        pltpu.VMEM((GROUP_ROWS, N), jnp.float32),
        pltpu.VMEM((LMAX,), jnp.float32),
        pltpu.VMEM((LMAX,), jnp.float32),
        pltpu.VMEM((LMAX,), jnp.int32),
        pltpu.VMEM((LMAX,), jnp.int32),
        pltpu.SemaphoreType.DMA,
        pltpu.SemaphoreType.DMA,
        pltpu.SemaphoreType.DMA,
        pltpu.SemaphoreType.DMA,
        pltpu.SemaphoreType.DMA,
    ],
)
def _triu_sc(x_ref, idx_ref, out_ref, slab0, slab1, buf0, buf1, idxA, idxB,
             ss0, ss1, ts0, ts1, is0):
    wid = lax.axis_index("s") * NC + lax.axis_index("c")
    base = wid * BATCH_PER_W
    idxbufs = (idxA, idxB)

    def idx_cp(g, buf):
        return pltpu.make_async_copy(
            idx_ref.at[pl.ds(POFF[g], PLEN[g])], buf.at[pl.ds(0, PLEN[g])], is0)

    def slab_cp(g, b, sl, sem):
        w = GWID[g]
        return pltpu.make_async_copy(
            x_ref.at[b, pl.ds(g * GROUP_ROWS, GROUP_ROWS), pl.ds(g * GROUP_ROWS, w)],
            sl.at[:, pl.ds(0, w)], sem)

    idx_cp(0, idxA).start()
    idx_cp(0, idxA).wait()
    slab_cp(0, base, slab0, ss0).start()

    for g in range(NGROUPS):
        goff, glen, plen = GOFF[g], GLEN[g], PLEN[g]
        idxg = idxbufs[g % 2]

        def out_cp(b, buf, sem, goff=goff, glen=glen):
            return pltpu.make_async_copy(
                buf.at[pl.ds(0, glen)],
                out_ref.at[pl.ds(b * OUT_LEN + goff, glen)], sem)

        def gather(slab, buf, plen=plen, idxg=idxg):
            @plsc.parallel_loop(0, plen, 64, unroll=4)
            def _(o):
                for k in range(4):
                    ids = idxg[pl.ds(o + k * 16, 16)]
                    rows = lax.shift_right_logical(ids, 9)
                    cols = lax.bitwise_and(ids, N - 1)
                    buf[pl.ds(o + k * 16, 16)] = plsc.load_gather(
                        slab, [rows, cols])

        def pair_body(i2, _, g=g):
            i = i2 * 2
            ba = base + i
            slab_cp(g, ba, slab0, ss0).wait()
            slab_cp(g, ba + 1, slab1, ss1).start()

            @pl.when(i > 0)
            def _():
                out_cp(ba - 2, buf0, ts0).wait()

            gather(slab0, buf0)
            out_cp(ba, buf0, ts0).start()

            slab_cp(g, ba + 1, slab1, ss1).wait()

            @pl.when(i < BATCH_PER_W - 2)
            def _():
                slab_cp(g, ba + 2, slab0, ss0).start()

            @pl.when(i > 0)
            def _():
                out_cp(ba - 1, buf1, ts1).wait()

            gather(slab1, buf1)
            out_cp(ba + 1, buf1, ts1).start()
            return 0

        lax.fori_loop(0, BATCH_PER_W // 2, pair_body, 0, unroll=False)

        if g + 1 < NGROUPS:
            idx_cp(g + 1, idxbufs[(g + 1) % 2]).start()
            slab_cp(g + 1, base, slab0, ss0).start()
            idx_cp(g + 1, idxbufs[(g + 1) % 2]).wait()

        out_cp(base + BATCH_PER_W - 2, buf0, ts0).wait()
        out_cp(base + BATCH_PER_W - 1, buf1, ts1).wait()


_OFFR = [r * N - r * (r - 1) // 2 for r in range(N + 1)]


_TC_PANELS = [
    (0, 128, 0, 512),
    (128, 128, 128, 128),
    (128, 128, 256, 256),
    (256, 128, 256, 256),
    (384, 128, 384, 128),
]


_TC_MB = 2


def _tc_body(*refs):
    x_refs, out_ref = refs[:-1], refs[-1]
    for bb in range(_TC_MB):
        for r in range(N - 1, -1, -1):
            for x_ref, (r0, nr, c0, nc) in zip(x_refs, _TC_PANELS):
                if not (r0 <= r < r0 + nr) or c0 + nc <= r:
                    continue
                out_ref[bb, 0, pl.ds(_OFFR[r] - r + c0, nc)] = (
                    x_ref[bb, r - r0, :])


def _triu_tc(xs):
    nb = xs.shape[0]

    def spec(r0, nr, c0, nc):
        return pl.BlockSpec(
            (_TC_MB, nr, nc), lambda b, r0=r0, nr=nr, c0=c0, nc=nc:
            (b, r0 // nr, c0 // nc))

    out = pl.pallas_call(
        _tc_body,
        grid=(nb // _TC_MB,),
        in_specs=[spec(*p) for p in _TC_PANELS],
        out_specs=pl.BlockSpec((_TC_MB, 1, OUT_LEN), lambda b: (b, 0, 0)),
        out_shape=jax.ShapeDtypeStruct((nb, 1, OUT_LEN), jnp.float32),
    )(*([xs] * len(_TC_PANELS)))
    return out.reshape(nb, OUT_LEN)


def kernel(x):
    return _triu_tc(x)


# speedup vs baseline: 1.3498x; 1.1087x over previous
import functools

import numpy as np
import jax
import jax.numpy as jnp
from jax import lax
from jax.experimental import pallas as pl
from jax.experimental.pallas import tpu as pltpu
from jax.experimental.pallas import tpu_sc as plsc

B = 256
N = 512
OUT_LEN = N * (N + 1) // 2
GROUP_ROWS = 32
NGROUPS = N // GROUP_ROWS

NC, NS = 2, 16
NW = NC * NS
BATCH_PER_W = B // NW

_r, _c = np.triu_indices(N)
_g = _r // GROUP_ROWS

GLEN = [int(np.sum(_g == g)) for g in range(NGROUPS)]
GOFF = [int(np.searchsorted(_g, g)) for g in range(NGROUPS)]
PLEN = [-(-l // 64) * 64 for l in GLEN]
POFF = list(np.cumsum([0] + PLEN[:-1]))
LMAX = PLEN[0]
GWID = [N - GROUP_ROWS * g for g in range(NGROUPS)]

_local = (_r - _g * GROUP_ROWS) * N + (_c - _g * GROUP_ROWS)
_idx_parts = []
for _gg in range(NGROUPS):
    _part = _local[GOFF[_gg]:GOFF[_gg] + GLEN[_gg]]
    _idx_parts.append(np.pad(_part, (0, PLEN[_gg] - GLEN[_gg])))
IDX_LOCAL = np.concatenate(_idx_parts).astype(np.int32)


@functools.partial(
    pl.kernel,
    mesh=plsc.VectorSubcoreMesh(core_axis_name="c", subcore_axis_name="s"),
    out_type=jax.ShapeDtypeStruct((B * OUT_LEN,), jnp.float32),
    compiler_params=pltpu.CompilerParams(
        needs_layout_passes=False, use_tc_tiling_on_sc=False),
    scratch_types=[
        pltpu.VMEM((GROUP_ROWS, N), jnp.float32),
        pltpu.VMEM((GROUP_ROWS, N), jnp.float32),
        pltpu.VMEM((LMAX,), jnp.float32),
        pltpu.VMEM((LMAX,), jnp.float32),
        pltpu.VMEM((LMAX,), jnp.int32),
        pltpu.VMEM((LMAX,), jnp.int32),
        pltpu.SemaphoreType.DMA,
        pltpu.SemaphoreType.DMA,
        pltpu.SemaphoreType.DMA,
        pltpu.SemaphoreType.DMA,
        pltpu.SemaphoreType.DMA,
    ],
)
def _triu_sc(x_ref, idx_ref, out_ref, slab0, slab1, buf0, buf1, idxA, idxB,
             ss0, ss1, ts0, ts1, is0):
    wid = lax.axis_index("s") * NC + lax.axis_index("c")
    base = wid * BATCH_PER_W
    idxbufs = (idxA, idxB)

    def idx_cp(g, buf):
        return pltpu.make_async_copy(
            idx_ref.at[pl.ds(POFF[g], PLEN[g])], buf.at[pl.ds(0, PLEN[g])], is0)

    def slab_cp(g, b, sl, sem):
        w = GWID[g]
        return pltpu.make_async_copy(
            x_ref.at[b, pl.ds(g * GROUP_ROWS, GROUP_ROWS), pl.ds(g * GROUP_ROWS, w)],
            sl.at[:, pl.ds(0, w)], sem)

    idx_cp(0, idxA).start()
    idx_cp(0, idxA).wait()
    slab_cp(0, base, slab0, ss0).start()

    for g in range(NGROUPS):
        goff, glen, plen = GOFF[g], GLEN[g], PLEN[g]
        idxg = idxbufs[g % 2]

        def out_cp(b, buf, sem, goff=goff, glen=glen):
            return pltpu.make_async_copy(
                buf.at[pl.ds(0, glen)],
                out_ref.at[pl.ds(b * OUT_LEN + goff, glen)], sem)

        def gather(slab, buf, plen=plen, idxg=idxg):
            @plsc.parallel_loop(0, plen, 64, unroll=4)
            def _(o):
                for k in range(4):
                    ids = idxg[pl.ds(o + k * 16, 16)]
                    rows = lax.shift_right_logical(ids, 9)
                    cols = lax.bitwise_and(ids, N - 1)
                    buf[pl.ds(o + k * 16, 16)] = plsc.load_gather(
                        slab, [rows, cols])

        def pair_body(i2, _, g=g):
            i = i2 * 2
            ba = base + i
            slab_cp(g, ba, slab0, ss0).wait()
            slab_cp(g, ba + 1, slab1, ss1).start()

            @pl.when(i > 0)
            def _():
                out_cp(ba - 2, buf0, ts0).wait()

            gather(slab0, buf0)
            out_cp(ba, buf0, ts0).start()

            slab_cp(g, ba + 1, slab1, ss1).wait()

            @pl.when(i < BATCH_PER_W - 2)
            def _():
                slab_cp(g, ba + 2, slab0, ss0).start()

            @pl.when(i > 0)
            def _():
                out_cp(ba - 1, buf1, ts1).wait()

            gather(slab1, buf1)
            out_cp(ba + 1, buf1, ts1).start()
            return 0

        lax.fori_loop(0, BATCH_PER_W // 2, pair_body, 0, unroll=False)

        if g + 1 < NGROUPS:
            idx_cp(g + 1, idxbufs[(g + 1) % 2]).start()
            slab_cp(g + 1, base, slab0, ss0).start()
            idx_cp(g + 1, idxbufs[(g + 1) % 2]).wait()

        out_cp(base + BATCH_PER_W - 2, buf0, ts0).wait()
        out_cp(base + BATCH_PER_W - 1, buf1, ts1).wait()


_OFFR = [r * N - r * (r - 1) // 2 for r in range(N + 1)]


_TC_PANELS = [
    (0, 128, 0, 512),
    (128, 128, 128, 128),
    (128, 128, 256, 256),
    (256, 128, 256, 256),
    (384, 128, 384, 128),
]


_TC_MB = 4


def _tc_body(*refs):
    x_refs, out_ref = refs[:-1], refs[-1]
    for bb in range(_TC_MB):
        for r in range(N - 1, -1, -1):
            for x_ref, (r0, nr, c0, nc) in zip(x_refs, _TC_PANELS):
                if not (r0 <= r < r0 + nr) or c0 + nc <= r:
                    continue
                out_ref[bb, 0, pl.ds(_OFFR[r] - r + c0, nc)] = (
                    x_ref[bb, r - r0, :])


def _triu_tc(xs):
    nb = xs.shape[0]

    def spec(r0, nr, c0, nc):
        return pl.BlockSpec(
            (_TC_MB, nr, nc), lambda b, r0=r0, nr=nr, c0=c0, nc=nc:
            (b, r0 // nr, c0 // nc))

    out = pl.pallas_call(
        _tc_body,
        grid=(nb // _TC_MB,),
        in_specs=[spec(*p) for p in _TC_PANELS],
        out_specs=pl.BlockSpec((_TC_MB, 1, OUT_LEN), lambda b: (b, 0, 0)),
        out_shape=jax.ShapeDtypeStruct((nb, 1, OUT_LEN), jnp.float32),
    )(*([xs] * len(_TC_PANELS)))
    return out.reshape(nb, OUT_LEN)


def kernel(x):
    return _triu_tc(x)


# speedup vs baseline: 1.3756x; 1.0191x over previous
import functools

import numpy as np
import jax
import jax.numpy as jnp
from jax import lax
from jax.experimental import pallas as pl
from jax.experimental.pallas import tpu as pltpu
from jax.experimental.pallas import tpu_sc as plsc

B = 256
N = 512
OUT_LEN = N * (N + 1) // 2
GROUP_ROWS = 32
NGROUPS = N // GROUP_ROWS

NC, NS = 2, 16
NW = NC * NS
BATCH_PER_W = B // NW

_r, _c = np.triu_indices(N)
_g = _r // GROUP_ROWS

GLEN = [int(np.sum(_g == g)) for g in range(NGROUPS)]
GOFF = [int(np.searchsorted(_g, g)) for g in range(NGROUPS)]
PLEN = [-(-l // 64) * 64 for l in GLEN]
POFF = list(np.cumsum([0] + PLEN[:-1]))
LMAX = PLEN[0]
GWID = [N - GROUP_ROWS * g for g in range(NGROUPS)]

_local = (_r - _g * GROUP_ROWS) * N + (_c - _g * GROUP_ROWS)
_idx_parts = []
for _gg in range(NGROUPS):
    _part = _local[GOFF[_gg]:GOFF[_gg] + GLEN[_gg]]
    _idx_parts.append(np.pad(_part, (0, PLEN[_gg] - GLEN[_gg])))
IDX_LOCAL = np.concatenate(_idx_parts).astype(np.int32)


@functools.partial(
    pl.kernel,
    mesh=plsc.VectorSubcoreMesh(core_axis_name="c", subcore_axis_name="s"),
    out_type=jax.ShapeDtypeStruct((B * OUT_LEN,), jnp.float32),
    compiler_params=pltpu.CompilerParams(
        needs_layout_passes=False, use_tc_tiling_on_sc=False),
    scratch_types=[
        pltpu.VMEM((GROUP_ROWS, N), jnp.float32),
        pltpu.VMEM((GROUP_ROWS, N), jnp.float32),
        pltpu.VMEM((LMAX,), jnp.float32),
        pltpu.VMEM((LMAX,), jnp.float32),
        pltpu.VMEM((LMAX,), jnp.int32),
        pltpu.VMEM((LMAX,), jnp.int32),
        pltpu.SemaphoreType.DMA,
        pltpu.SemaphoreType.DMA,
        pltpu.SemaphoreType.DMA,
        pltpu.SemaphoreType.DMA,
        pltpu.SemaphoreType.DMA,
    ],
)
def _triu_sc(x_ref, idx_ref, out_ref, slab0, slab1, buf0, buf1, idxA, idxB,
             ss0, ss1, ts0, ts1, is0):
    wid = lax.axis_index("s") * NC + lax.axis_index("c")
    base = wid * BATCH_PER_W
    idxbufs = (idxA, idxB)

    def idx_cp(g, buf):
        return pltpu.make_async_copy(
            idx_ref.at[pl.ds(POFF[g], PLEN[g])], buf.at[pl.ds(0, PLEN[g])], is0)

    def slab_cp(g, b, sl, sem):
        w = GWID[g]
        return pltpu.make_async_copy(
            x_ref.at[b, pl.ds(g * GROUP_ROWS, GROUP_ROWS), pl.ds(g * GROUP_ROWS, w)],
            sl.at[:, pl.ds(0, w)], sem)

    idx_cp(0, idxA).start()
    idx_cp(0, idxA).wait()
    slab_cp(0, base, slab0, ss0).start()

    for g in range(NGROUPS):
        goff, glen, plen = GOFF[g], GLEN[g], PLEN[g]
        idxg = idxbufs[g % 2]

        def out_cp(b, buf, sem, goff=goff, glen=glen):
            return pltpu.make_async_copy(
                buf.at[pl.ds(0, glen)],
                out_ref.at[pl.ds(b * OUT_LEN + goff, glen)], sem)

        def gather(slab, buf, plen=plen, idxg=idxg):
            @plsc.parallel_loop(0, plen, 64, unroll=4)
            def _(o):
                for k in range(4):
                    ids = idxg[pl.ds(o + k * 16, 16)]
                    rows = lax.shift_right_logical(ids, 9)
                    cols = lax.bitwise_and(ids, N - 1)
                    buf[pl.ds(o + k * 16, 16)] = plsc.load_gather(
                        slab, [rows, cols])

        def pair_body(i2, _, g=g):
            i = i2 * 2
            ba = base + i
            slab_cp(g, ba, slab0, ss0).wait()
            slab_cp(g, ba + 1, slab1, ss1).start()

            @pl.when(i > 0)
            def _():
                out_cp(ba - 2, buf0, ts0).wait()

            gather(slab0, buf0)
            out_cp(ba, buf0, ts0).start()

            slab_cp(g, ba + 1, slab1, ss1).wait()

            @pl.when(i < BATCH_PER_W - 2)
            def _():
                slab_cp(g, ba + 2, slab0, ss0).start()

            @pl.when(i > 0)
            def _():
                out_cp(ba - 1, buf1, ts1).wait()

            gather(slab1, buf1)
            out_cp(ba + 1, buf1, ts1).start()
            return 0

        lax.fori_loop(0, BATCH_PER_W // 2, pair_body, 0, unroll=False)

        if g + 1 < NGROUPS:
            idx_cp(g + 1, idxbufs[(g + 1) % 2]).start()
            slab_cp(g + 1, base, slab0, ss0).start()
            idx_cp(g + 1, idxbufs[(g + 1) % 2]).wait()

        out_cp(base + BATCH_PER_W - 2, buf0, ts0).wait()
        out_cp(base + BATCH_PER_W - 1, buf1, ts1).wait()


_OFFR = [r * N - r * (r - 1) // 2 for r in range(N + 1)]


_TC_PANELS = [
    (0, 128, 0, 512),
    (128, 128, 128, 128),
    (128, 128, 256, 256),
    (256, 128, 256, 256),
    (384, 128, 384, 128),
]


_TC_MB = 8


def _tc_body(*refs):
    x_refs, out_ref = refs[:-1], refs[-1]
    for bb in range(_TC_MB):
        for r in range(N - 1, -1, -1):
            for x_ref, (r0, nr, c0, nc) in zip(x_refs, _TC_PANELS):
                if not (r0 <= r < r0 + nr) or c0 + nc <= r:
                    continue
                out_ref[bb, 0, pl.ds(_OFFR[r] - r + c0, nc)] = (
                    x_ref[bb, r - r0, :])


def _triu_tc(xs):
    nb = xs.shape[0]

    def spec(r0, nr, c0, nc):
        return pl.BlockSpec(
            (_TC_MB, nr, nc), lambda b, r0=r0, nr=nr, c0=c0, nc=nc:
            (b, r0 // nr, c0 // nc))

    out = pl.pallas_call(
        _tc_body,
        grid=(nb // _TC_MB,),
        in_specs=[spec(*p) for p in _TC_PANELS],
        out_specs=pl.BlockSpec((_TC_MB, 1, OUT_LEN), lambda b: (b, 0, 0)),
        out_shape=jax.ShapeDtypeStruct((nb, 1, OUT_LEN), jnp.float32),
    )(*([xs] * len(_TC_PANELS)))
    return out.reshape(nb, OUT_LEN)


def kernel(x):
    return _triu_tc(x)
